# baseline (speedup 1.0000x reference)
import jax
import jax.numpy as jnp
from jax import lax
from jax.experimental import pallas as pl


def _dg(a, w):
    return lax.dot_general(a, w, (((1,), (1,)), ((), ())),
                           preferred_element_type=jnp.float32)


def _fused_body(x_ref, w1_ref, b1_ref, wrA_ref, wrB_ref,
                wl0_ref, wl1_ref, wl2_ref, wl3_ref, cA_ref, cB_ref,
                w2_ref, wf_ref, k_ref, y_ref):
    xb = x_ref[0]
    h = jnp.dot(w1_ref[...], xb,
                preferred_element_type=jnp.float32) + b1_ref[...]
    hA = h[:64, :]
    hB = h[64:, :]
    mA = jnp.mean(hA, axis=0, keepdims=True)
    mB = jnp.mean(hB, axis=0, keepdims=True)
    msgA = 0.5 * (_dg(mB, wl1_ref[...]) + _dg(mA, wl2_ref[...])) + cA_ref[...]
    msgB = 0.5 * (_dg(mA, wl0_ref[...]) + _dg(mB, wl3_ref[...])) + cB_ref[...]
    preA = _dg(hA, wrA_ref[...]) + msgA
    preB = _dg(hB, wrB_ref[...]) + msgB
    r = jnp.maximum(jnp.concatenate([preA, preB], axis=0), 0.0)
    t = jnp.dot(w2_ref[...], r, preferred_element_type=jnp.float32)
    y_ref[0] = _dg(t, wf_ref[...]) + k_ref[...]


def kernel(x, W1, b1, W2, b2, sage_Wl, sage_bl, sage_Wr, Wf, bf, period):
    Bb, d_model, Lp, Pp = x.shape
    F = Lp * Pp
    x2 = x.reshape(Bb, d_model, F)

    wrA = 0.5 * (sage_Wr[1] + sage_Wr[2])
    wrB = 0.5 * (sage_Wr[0] + sage_Wr[3])
    cA = (0.5 * (sage_bl[1] + sage_bl[2]))[None, :]
    cB = (0.5 * (sage_bl[0] + sage_bl[3]))[None, :]
    k = jnp.sum(W2, axis=1)[:, None] * bf[None, :] + b2[:, None]
    b1c = b1[:, None]

    wspec = lambda shp: pl.BlockSpec(shp, lambda b: (0,) * len(shp))
    y2 = pl.pallas_call(
        _fused_body,
        grid=(Bb,),
        in_specs=[
            pl.BlockSpec((1, d_model, F), lambda b: (b, 0, 0)),
            wspec(W1.shape),
            wspec(b1c.shape),
            wspec(wrA.shape),
            wspec(wrB.shape),
            wspec(sage_Wl[0].shape),
            wspec(sage_Wl[1].shape),
            wspec(sage_Wl[2].shape),
            wspec(sage_Wl[3].shape),
            wspec(cA.shape),
            wspec(cB.shape),
            wspec(W2.shape),
            wspec(Wf.shape),
            wspec(k.shape),
        ],
        out_specs=pl.BlockSpec((1, W2.shape[0], F), lambda b: (b, 0, 0)),
        out_shape=jax.ShapeDtypeStruct((Bb, W2.shape[0], F), jnp.float32),
    )(x2, W1, b1c, wrA, wrB,
      sage_Wl[0], sage_Wl[1], sage_Wl[2], sage_Wl[3], cA, cB, W2, Wf, k)
    return y2.reshape(Bb, W2.shape[0], Lp, Pp)


# speedup vs baseline: 1.6388x; 1.6388x over previous
import jax
import jax.numpy as jnp
from jax import lax
from jax.experimental import pallas as pl


def _dg(a, w):
    return lax.dot_general(a, w, (((1,), (1,)), ((), ())),
                           preferred_element_type=jnp.float32)


_BT = 8


def _fused_body(x_ref, w1_ref, b1_ref, wrA_ref, wrB_ref,
                wl0_ref, wl1_ref, wl2_ref, wl3_ref, cA_ref, cB_ref,
                w2_ref, wf_ref, k_ref, y_ref):
    w1 = w1_ref[...]
    b1 = b1_ref[...]
    wrA = wrA_ref[...]
    wrB = wrB_ref[...]
    wl0, wl1, wl2, wl3 = wl0_ref[...], wl1_ref[...], wl2_ref[...], wl3_ref[...]
    cA, cB = cA_ref[...], cB_ref[...]
    w2 = w2_ref[...]
    wf = wf_ref[...]
    k = k_ref[...]
    for j in range(_BT):
        xb = x_ref[j]
        h = jnp.dot(w1, xb, preferred_element_type=jnp.float32) + b1
        hA = h[:64, :]
        hB = h[64:, :]
        mA = jnp.mean(hA, axis=0, keepdims=True)
        mB = jnp.mean(hB, axis=0, keepdims=True)
        msgA = 0.5 * (_dg(mB, wl1) + _dg(mA, wl2)) + cA
        msgB = 0.5 * (_dg(mA, wl0) + _dg(mB, wl3)) + cB
        preA = _dg(hA, wrA) + msgA
        preB = _dg(hB, wrB) + msgB
        r = jnp.maximum(jnp.concatenate([preA, preB], axis=0), 0.0)
        t = jnp.dot(w2, r, preferred_element_type=jnp.float32)
        y_ref[j] = _dg(t, wf) + k


def kernel(x, W1, b1, W2, b2, sage_Wl, sage_bl, sage_Wr, Wf, bf, period):
    Bb, d_model, Lp, Pp = x.shape
    F = Lp * Pp
    x2 = x.reshape(Bb, d_model, F)

    wrA = 0.5 * (sage_Wr[1] + sage_Wr[2])
    wrB = 0.5 * (sage_Wr[0] + sage_Wr[3])
    cA = (0.5 * (sage_bl[1] + sage_bl[2]))[None, :]
    cB = (0.5 * (sage_bl[0] + sage_bl[3]))[None, :]
    k = jnp.sum(W2, axis=1)[:, None] * bf[None, :] + b2[:, None]
    b1c = b1[:, None]

    wspec = lambda shp: pl.BlockSpec(shp, lambda b: (0,) * len(shp))
    y2 = pl.pallas_call(
        _fused_body,
        grid=(Bb // _BT,),
        in_specs=[
            pl.BlockSpec((_BT, d_model, F), lambda b: (b, 0, 0)),
            wspec(W1.shape),
            wspec(b1c.shape),
            wspec(wrA.shape),
            wspec(wrB.shape),
            wspec(sage_Wl[0].shape),
            wspec(sage_Wl[1].shape),
            wspec(sage_Wl[2].shape),
            wspec(sage_Wl[3].shape),
            wspec(cA.shape),
            wspec(cB.shape),
            wspec(W2.shape),
            wspec(Wf.shape),
            wspec(k.shape),
        ],
        out_specs=pl.BlockSpec((_BT, W2.shape[0], F), lambda b: (b, 0, 0)),
        out_shape=jax.ShapeDtypeStruct((Bb, W2.shape[0], F), jnp.float32),
    )(x2, W1, b1c, wrA, wrB,
      sage_Wl[0], sage_Wl[1], sage_Wl[2], sage_Wl[3], cA, cB, W2, Wf, k)
    return y2.reshape(Bb, W2.shape[0], Lp, Pp)


# speedup vs baseline: 2.5845x; 1.5771x over previous
import jax
import jax.numpy as jnp
from jax import lax
from jax.experimental import pallas as pl


def _dg(a, w):
    return lax.dot_general(a, w, (((1,), (1,)), ((), ())),
                           preferred_element_type=jnp.float32)


_BT = 8


def _fused_body(x_ref, w1_ref, b1_ref, wrA_ref, wrB_ref,
                wl0_ref, wl1_ref, wl2_ref, wl3_ref, cA_ref, cB_ref,
                w2_ref, wf_ref, k_ref, y_ref):
    w1 = w1_ref[...]
    b1 = b1_ref[...]
    wrA = wrA_ref[...]
    wrB = wrB_ref[...]
    wl0, wl1, wl2, wl3 = wl0_ref[...], wl1_ref[...], wl2_ref[...], wl3_ref[...]
    cA, cB = cA_ref[...], cB_ref[...]
    w2 = w2_ref[...]
    wf = wf_ref[...]
    k = k_ref[...]
    hs = [jnp.dot(w1, x_ref[j], preferred_element_type=jnp.float32) + b1
          for j in range(_BT)]
    mAs = [jnp.mean(h[:64, :], axis=0, keepdims=True) for h in hs]
    mBs = [jnp.mean(h[64:, :], axis=0, keepdims=True) for h in hs]
    msgAs = [0.5 * (_dg(mBs[j], wl1) + _dg(mAs[j], wl2)) + cA
             for j in range(_BT)]
    msgBs = [0.5 * (_dg(mAs[j], wl0) + _dg(mBs[j], wl3)) + cB
             for j in range(_BT)]
    preAs = [_dg(hs[j][:64, :], wrA) + msgAs[j] for j in range(_BT)]
    preBs = [_dg(hs[j][64:, :], wrB) + msgBs[j] for j in range(_BT)]
    rs = [jnp.maximum(jnp.concatenate([preAs[j], preBs[j]], axis=0), 0.0)
          for j in range(_BT)]
    ts = [jnp.dot(w2, r, preferred_element_type=jnp.float32) for r in rs]
    for j in range(_BT):
        y_ref[j] = _dg(ts[j], wf) + k


def kernel(x, W1, b1, W2, b2, sage_Wl, sage_bl, sage_Wr, Wf, bf, period):
    Bb, d_model, Lp, Pp = x.shape
    F = Lp * Pp
    x2 = x.reshape(Bb, d_model, F)

    wrA = 0.5 * (sage_Wr[1] + sage_Wr[2])
    wrB = 0.5 * (sage_Wr[0] + sage_Wr[3])
    cA = (0.5 * (sage_bl[1] + sage_bl[2]))[None, :]
    cB = (0.5 * (sage_bl[0] + sage_bl[3]))[None, :]
    k = jnp.sum(W2, axis=1)[:, None] * bf[None, :] + b2[:, None]
    b1c = b1[:, None]

    wspec = lambda shp: pl.BlockSpec(shp, lambda b: (0,) * len(shp))
    y2 = pl.pallas_call(
        _fused_body,
        grid=(Bb // _BT,),
        in_specs=[
            pl.BlockSpec((_BT, d_model, F), lambda b: (b, 0, 0)),
            wspec(W1.shape),
            wspec(b1c.shape),
            wspec(wrA.shape),
            wspec(wrB.shape),
            wspec(sage_Wl[0].shape),
            wspec(sage_Wl[1].shape),
            wspec(sage_Wl[2].shape),
            wspec(sage_Wl[3].shape),
            wspec(cA.shape),
            wspec(cB.shape),
            wspec(W2.shape),
            wspec(Wf.shape),
            wspec(k.shape),
        ],
        out_specs=pl.BlockSpec((_BT, W2.shape[0], F), lambda b: (b, 0, 0)),
        out_shape=jax.ShapeDtypeStruct((Bb, W2.shape[0], F), jnp.float32),
    )(x2, W1, b1c, wrA, wrB,
      sage_Wl[0], sage_Wl[1], sage_Wl[2], sage_Wl[3], cA, cB, W2, Wf, k)
    return y2.reshape(Bb, W2.shape[0], Lp, Pp)
